# Initial kernel scaffold; baseline (speedup 1.0000x reference)
import jax
import jax.numpy as jnp
from jax.experimental import pallas as pl


def kernel(values, row_idx, col_idx, W_all, b_all, W_n, b_n, W_m, b_m, W_both, b_both):
    raise NotImplementedError("write your pallas kernel here")


# speedup vs baseline: 1.0725x; 1.0725x over previous
import functools

import jax
import jax.numpy as jnp
from jax import lax
from jax.experimental import pallas as pl
from jax.experimental.pallas import tpu as pltpu
from jax.experimental.pallas import tpu_sc as plsc

N_CAMS = 500
N_PTS = 50000
NNZ = 500000
D = 128

PTS_PAD = 50048
CAMS_PAD = 512
NNZ_BLOCKS = 977
NNZ_RPAD = NNZ_BLOCKS * 512
IDX_ROWS = 3968
NNZ_PAD = IDX_ROWS * 128
ROW_DUMP = 504

P2_ROWS_PER_TILE = IDX_ROWS // 32
P2_BLOCKS = 62

_MESH = plsc.VectorSubcoreMesh(core_axis_name="c", subcore_axis_name="s")
_HI = lax.Precision.HIGHEST


@functools.partial(
    pl.kernel,
    out_type=jax.ShapeDtypeStruct((NNZ_PAD, D), jnp.float32),
    mesh=_MESH,
    scratch_types=dict(
        cbuf=pltpu.VMEM((2, 128), jnp.int32),
        abuf=pltpu.VMEM((256, D), jnp.float32),
        sem=pltpu.SemaphoreType.DMA,
    ),
)
def _gather_a(a_tbl, cidx2, gath, *, cbuf, abuf, sem):
    c = lax.axis_index("c")
    s = lax.axis_index("s")
    w = s * 2 + c

    def blk(k, _):
        row_base = w * P2_ROWS_PER_TILE + k * 2
        nnz_base = row_base * 128
        pltpu.sync_copy(cidx2.at[pl.ds(row_base, 2)], cbuf)
        cps = [
            pltpu.async_copy(a_tbl.at[cbuf.at[j]],
                             abuf.at[pl.ds(j * 128, 128)], sem)
            for j in range(2)
        ]
        for cp in cps:
            cp.wait()
        pltpu.sync_copy(abuf, gath.at[pl.ds(nnz_base, 256)])
        return 0

    lax.fori_loop(0, P2_BLOCKS, blk, 0)


def _rows_body(v_ref, rid_ref, rs_ref, rc_ref):
    i = pl.program_id(0)

    @pl.when(i == 0)
    def _():
        rs_ref[...] = jnp.zeros((CAMS_PAD, D), jnp.float32)
        rc_ref[...] = jnp.zeros((CAMS_PAD, D), jnp.float32)

    valid = (lax.broadcasted_iota(jnp.int32, (512, 1), 0) + i * 512) < NNZ
    v = jnp.where(valid, v_ref[...], 0.0)
    rid = rid_ref[...][0, 0]
    oh = (rid[:, None]
          == lax.broadcasted_iota(jnp.int32, (512, CAMS_PAD), 1))
    oh = jnp.where(valid, oh.astype(jnp.float32), 0.0)
    dn = (((0,), (0,)), ((), ()))
    rs_ref[...] += lax.dot_general(oh, v, dn, precision=_HI)
    rc_ref[...] += lax.dot_general(oh, jnp.ones((512, D), jnp.float32),
                                   dn, precision=_HI)


def _tables_body(cm_ref, rs_ref, rc_ref, wn_ref, wm_ref, wb_ref,
                 bsum_ref, bm_ref, a_ref, b_ref):
    rmask = lax.broadcasted_iota(jnp.int32, (CAMS_PAD, 1), 0) < N_CAMS
    rs = jnp.where(rmask, rs_ref[...], 0.0)
    g = jnp.sum(rs, axis=0, keepdims=True)
    go = jnp.dot(g, wb_ref[...], precision=_HI)
    const = 0.25 * (bsum_ref[...] + go * (1.0 / NNZ))
    rmean = rs / jnp.maximum(rc_ref[...], 1.0)
    bmat = 0.25 * (jnp.dot(rmean, wm_ref[...], precision=_HI) + bm_ref[...])
    b_ref[...] = jnp.where(rmask, bmat, 0.0)
    a_ref[...] = (0.25 * jnp.dot(cm_ref[...], wn_ref[...], precision=_HI)
                  + const)


def _final_body(v_ref, g_ref, rid_ref, wa_ref, b_ref, out_ref):
    rid = rid_ref[...][0, 0]
    oh = (rid[:, None]
          == lax.broadcasted_iota(jnp.int32, (512, CAMS_PAD), 1))
    oh = oh.astype(jnp.float32)
    out_ref[...] = (0.25 * jnp.dot(v_ref[...], wa_ref[...], precision=_HI)
                    + g_ref[...]
                    + jnp.dot(oh, b_ref[...], precision=_HI))


def kernel(values, row_idx, col_idx, W_all, b_all, W_n, b_n, W_m, b_m,
           W_both, b_both):
    f32 = jnp.float32
    row_idx = row_idx.astype(jnp.int32)
    col_idx = col_idx.astype(jnp.int32)

    rid3 = jnp.concatenate(
        [row_idx, jnp.full((NNZ_RPAD - NNZ,), ROW_DUMP, jnp.int32)]
    ).reshape(NNZ_BLOCKS, 1, 512)
    cidx2 = jnp.concatenate(
        [col_idx, jnp.zeros((NNZ_PAD - NNZ,), jnp.int32)]
    ).reshape(IDX_ROWS, 128)

    ccnt = jax.ops.segment_sum(jnp.ones((NNZ,), f32), col_idx,
                               num_segments=PTS_PAD)
    csum = jax.ops.segment_sum(values, col_idx, num_segments=PTS_PAD)
    cmean = csum / jnp.maximum(ccnt, 1.0)[:, None]

    row_sums, row_cntb = pl.pallas_call(
        _rows_body,
        grid=(NNZ_BLOCKS,),
        in_specs=[
            pl.BlockSpec((512, D), lambda i: (i, 0)),
            pl.BlockSpec((1, 1, 512), lambda i: (i, 0, 0)),
        ],
        out_specs=[
            pl.BlockSpec((CAMS_PAD, D), lambda i: (0, 0)),
            pl.BlockSpec((CAMS_PAD, D), lambda i: (0, 0)),
        ],
        out_shape=[
            jax.ShapeDtypeStruct((CAMS_PAD, D), f32),
            jax.ShapeDtypeStruct((CAMS_PAD, D), f32),
        ],
    )(values, rid3)

    bsum = (b_all + b_n + b_both).reshape(1, D)
    bm = b_m.reshape(1, D)

    a_tbl, b_tbl = pl.pallas_call(
        _tables_body,
        grid=(98,),
        in_specs=[
            pl.BlockSpec((512, D), lambda i: (i, 0)),
            pl.BlockSpec((CAMS_PAD, D), lambda i: (0, 0)),
            pl.BlockSpec((CAMS_PAD, D), lambda i: (0, 0)),
            pl.BlockSpec((D, D), lambda i: (0, 0)),
            pl.BlockSpec((D, D), lambda i: (0, 0)),
            pl.BlockSpec((D, D), lambda i: (0, 0)),
            pl.BlockSpec((1, D), lambda i: (0, 0)),
            pl.BlockSpec((1, D), lambda i: (0, 0)),
        ],
        out_specs=[
            pl.BlockSpec((512, D), lambda i: (i, 0)),
            pl.BlockSpec((CAMS_PAD, D), lambda i: (0, 0)),
        ],
        out_shape=[
            jax.ShapeDtypeStruct((PTS_PAD, D), f32),
            jax.ShapeDtypeStruct((CAMS_PAD, D), f32),
        ],
    )(cmean, row_sums, row_cntb, W_n, W_m, W_both, bsum, bm)

    gath = _gather_a(a_tbl, cidx2)

    out = pl.pallas_call(
        _final_body,
        grid=(NNZ_BLOCKS,),
        in_specs=[
            pl.BlockSpec((512, D), lambda i: (i, 0)),
            pl.BlockSpec((512, D), lambda i: (i, 0)),
            pl.BlockSpec((1, 1, 512), lambda i: (i, 0, 0)),
            pl.BlockSpec((D, D), lambda i: (0, 0)),
            pl.BlockSpec((CAMS_PAD, D), lambda i: (0, 0)),
        ],
        out_specs=pl.BlockSpec((512, D), lambda i: (i, 0)),
        out_shape=jax.ShapeDtypeStruct((NNZ, D), f32),
    )(values, gath, rid3, W_all, b_tbl)

    return out


# speedup vs baseline: 1.0947x; 1.0206x over previous
import functools

import jax
import jax.numpy as jnp
from jax import lax
from jax.experimental import pallas as pl
from jax.experimental.pallas import tpu as pltpu
from jax.experimental.pallas import tpu_sc as plsc

N_CAMS = 500
N_PTS = 50000
NNZ = 500000
D = 128

PTS_PAD = 50048
CAMS_PAD = 512
NNZ_BLOCKS = 977
NNZ_RPAD = NNZ_BLOCKS * 512
IDX_ROWS = 3968
NNZ_PAD = IDX_ROWS * 128
ROW_DUMP = 504

P2_ROWS_PER_TILE = IDX_ROWS // 32
P2_BLOCKS = 62

_MESH = plsc.VectorSubcoreMesh(core_axis_name="c", subcore_axis_name="s")
_HI = lax.Precision.HIGHEST


@functools.partial(
    pl.kernel,
    out_type=jax.ShapeDtypeStruct((NNZ_PAD, D), jnp.float32),
    mesh=_MESH,
    scratch_types=dict(
        cb0=pltpu.VMEM((2, 128), jnp.int32),
        cb1=pltpu.VMEM((2, 128), jnp.int32),
        ab0=pltpu.VMEM((256, D), jnp.float32),
        ab1=pltpu.VMEM((256, D), jnp.float32),
        sem0=pltpu.SemaphoreType.DMA,
        sem1=pltpu.SemaphoreType.DMA,
    ),
)
def _gather_a(a_tbl, cidx2, gath, *, cb0, cb1, ab0, ab1, sem0, sem1):
    c = lax.axis_index("c")
    s = lax.axis_index("s")
    w = s * 2 + c
    tile_row = w * P2_ROWS_PER_TILE

    def start(k, cb, ab, sem):
        pltpu.sync_copy(cidx2.at[pl.ds(tile_row + k * 2, 2)], cb)
        for j in range(2):
            pltpu.async_copy(a_tbl.at[cb.at[j]],
                             ab.at[pl.ds(j * 128, 128)], sem)

    def drain(k, cb, ab, sem):
        for j in range(2):
            pltpu.make_async_copy(a_tbl.at[cb.at[j]],
                                  ab.at[pl.ds(j * 128, 128)], sem).wait()
        pltpu.sync_copy(ab, gath.at[pl.ds((tile_row + k * 2) * 128, 256)])

    start(0, cb0, ab0, sem0)

    def pair(p, _):
        k0, k1 = 2 * p, 2 * p + 1
        start(k1, cb1, ab1, sem1)
        drain(k0, cb0, ab0, sem0)

        @pl.when(p < P2_BLOCKS // 2 - 1)
        def _():
            start(k0 + 2, cb0, ab0, sem0)

        drain(k1, cb1, ab1, sem1)
        return 0

    lax.fori_loop(0, P2_BLOCKS // 2, pair, 0)


def _rows_body(v_ref, rid_ref, rs_ref, rc_ref):
    i = pl.program_id(0)

    @pl.when(i == 0)
    def _():
        rs_ref[...] = jnp.zeros((CAMS_PAD, D), jnp.float32)
        rc_ref[...] = jnp.zeros((CAMS_PAD, D), jnp.float32)

    valid = (lax.broadcasted_iota(jnp.int32, (512, 1), 0) + i * 512) < NNZ
    v = jnp.where(valid, v_ref[...], 0.0)
    rid = rid_ref[...][0, 0]
    oh = (rid[:, None]
          == lax.broadcasted_iota(jnp.int32, (512, CAMS_PAD), 1))
    oh = jnp.where(valid, oh.astype(jnp.float32), 0.0)
    dn = (((0,), (0,)), ((), ()))
    rs_ref[...] += lax.dot_general(oh, v, dn, precision=_HI)
    rc_ref[...] += lax.dot_general(oh, jnp.ones((512, D), jnp.float32),
                                   dn, precision=_HI)


def _tables_body(cm_ref, rs_ref, rc_ref, wn_ref, wm_ref, wb_ref,
                 bsum_ref, bm_ref, a_ref, b_ref):
    rmask = lax.broadcasted_iota(jnp.int32, (CAMS_PAD, 1), 0) < N_CAMS
    rs = jnp.where(rmask, rs_ref[...], 0.0)
    g = jnp.sum(rs, axis=0, keepdims=True)
    go = jnp.dot(g, wb_ref[...], precision=_HI)
    const = 0.25 * (bsum_ref[...] + go * (1.0 / NNZ))
    rmean = rs / jnp.maximum(rc_ref[...], 1.0)
    bmat = 0.25 * (jnp.dot(rmean, wm_ref[...], precision=_HI) + bm_ref[...])
    b_ref[...] = jnp.where(rmask, bmat, 0.0)
    a_ref[...] = (0.25 * jnp.dot(cm_ref[...], wn_ref[...], precision=_HI)
                  + const)


def _final_body(v_ref, g_ref, rid_ref, wa_ref, b_ref, out_ref):
    rid = rid_ref[...][0, 0]
    oh = (rid[:, None]
          == lax.broadcasted_iota(jnp.int32, (512, CAMS_PAD), 1))
    oh = oh.astype(jnp.float32)
    out_ref[...] = (0.25 * jnp.dot(v_ref[...], wa_ref[...], precision=_HI)
                    + g_ref[...]
                    + jnp.dot(oh, b_ref[...], precision=_HI))


def kernel(values, row_idx, col_idx, W_all, b_all, W_n, b_n, W_m, b_m,
           W_both, b_both):
    f32 = jnp.float32
    row_idx = row_idx.astype(jnp.int32)
    col_idx = col_idx.astype(jnp.int32)

    rid3 = jnp.concatenate(
        [row_idx, jnp.full((NNZ_RPAD - NNZ,), ROW_DUMP, jnp.int32)]
    ).reshape(NNZ_BLOCKS, 1, 512)
    cidx2 = jnp.concatenate(
        [col_idx, jnp.zeros((NNZ_PAD - NNZ,), jnp.int32)]
    ).reshape(IDX_ROWS, 128)

    ccnt = jax.ops.segment_sum(jnp.ones((NNZ,), f32), col_idx,
                               num_segments=PTS_PAD)
    csum = jax.ops.segment_sum(values, col_idx, num_segments=PTS_PAD)
    cmean = csum / jnp.maximum(ccnt, 1.0)[:, None]

    row_sums, row_cntb = pl.pallas_call(
        _rows_body,
        grid=(NNZ_BLOCKS,),
        in_specs=[
            pl.BlockSpec((512, D), lambda i: (i, 0)),
            pl.BlockSpec((1, 1, 512), lambda i: (i, 0, 0)),
        ],
        out_specs=[
            pl.BlockSpec((CAMS_PAD, D), lambda i: (0, 0)),
            pl.BlockSpec((CAMS_PAD, D), lambda i: (0, 0)),
        ],
        out_shape=[
            jax.ShapeDtypeStruct((CAMS_PAD, D), f32),
            jax.ShapeDtypeStruct((CAMS_PAD, D), f32),
        ],
    )(values, rid3)

    bsum = (b_all + b_n + b_both).reshape(1, D)
    bm = b_m.reshape(1, D)

    a_tbl, b_tbl = pl.pallas_call(
        _tables_body,
        grid=(98,),
        in_specs=[
            pl.BlockSpec((512, D), lambda i: (i, 0)),
            pl.BlockSpec((CAMS_PAD, D), lambda i: (0, 0)),
            pl.BlockSpec((CAMS_PAD, D), lambda i: (0, 0)),
            pl.BlockSpec((D, D), lambda i: (0, 0)),
            pl.BlockSpec((D, D), lambda i: (0, 0)),
            pl.BlockSpec((D, D), lambda i: (0, 0)),
            pl.BlockSpec((1, D), lambda i: (0, 0)),
            pl.BlockSpec((1, D), lambda i: (0, 0)),
        ],
        out_specs=[
            pl.BlockSpec((512, D), lambda i: (i, 0)),
            pl.BlockSpec((CAMS_PAD, D), lambda i: (0, 0)),
        ],
        out_shape=[
            jax.ShapeDtypeStruct((PTS_PAD, D), f32),
            jax.ShapeDtypeStruct((CAMS_PAD, D), f32),
        ],
    )(cmean, row_sums, row_cntb, W_n, W_m, W_both, bsum, bm)

    gath = _gather_a(a_tbl, cidx2)

    out = pl.pallas_call(
        _final_body,
        grid=(NNZ_BLOCKS,),
        in_specs=[
            pl.BlockSpec((512, D), lambda i: (i, 0)),
            pl.BlockSpec((512, D), lambda i: (i, 0)),
            pl.BlockSpec((1, 1, 512), lambda i: (i, 0, 0)),
            pl.BlockSpec((D, D), lambda i: (0, 0)),
            pl.BlockSpec((CAMS_PAD, D), lambda i: (0, 0)),
        ],
        out_specs=pl.BlockSpec((512, D), lambda i: (i, 0)),
        out_shape=jax.ShapeDtypeStruct((NNZ, D), f32),
    )(values, gath, rid3, W_all, b_tbl)

    return out


# speedup vs baseline: 1.3826x; 1.2631x over previous
import functools

import jax
import jax.numpy as jnp
from jax import lax
from jax.experimental import pallas as pl
from jax.experimental.pallas import tpu as pltpu
from jax.experimental.pallas import tpu_sc as plsc

N_CAMS = 500
N_PTS = 50000
NNZ = 500000
D = 128

PTS_PAD = 50048
CAMS_PAD = 512
NNZ_BLOCKS = 977
NNZ_RPAD = NNZ_BLOCKS * 512
IDX_ROWS = 3968
NNZ_PAD = IDX_ROWS * 128
ROW_DUMP = 504

P2_ROWS_PER_TILE = IDX_ROWS // 32
P2_BLOCKS = 62

_MESH = plsc.VectorSubcoreMesh(core_axis_name="c", subcore_axis_name="s")
_HI = lax.Precision.HIGHEST


@functools.partial(
    pl.kernel,
    out_type=jax.ShapeDtypeStruct((NNZ_PAD, D), jnp.float32),
    mesh=_MESH,
    scratch_types=dict(
        cb0=pltpu.VMEM((2, 128), jnp.int32),
        cb1=pltpu.VMEM((2, 128), jnp.int32),
        ab0=pltpu.VMEM((256, D), jnp.float32),
        ab1=pltpu.VMEM((256, D), jnp.float32),
        sem0=pltpu.SemaphoreType.DMA,
        sem1=pltpu.SemaphoreType.DMA,
    ),
)
def _gather_a(a_tbl, cidx2, gath, *, cb0, cb1, ab0, ab1, sem0, sem1):
    c = lax.axis_index("c")
    s = lax.axis_index("s")
    w = s * 2 + c
    tile_row = w * P2_ROWS_PER_TILE

    def start(k, cb, ab, sem):
        pltpu.sync_copy(cidx2.at[pl.ds(tile_row + k * 2, 2)], cb)
        for j in range(2):
            pltpu.async_copy(a_tbl.at[cb.at[j]],
                             ab.at[pl.ds(j * 128, 128)], sem)

    def drain(k, cb, ab, sem):
        for j in range(2):
            pltpu.make_async_copy(a_tbl.at[cb.at[j]],
                                  ab.at[pl.ds(j * 128, 128)], sem).wait()
        pltpu.sync_copy(ab, gath.at[pl.ds((tile_row + k * 2) * 128, 256)])

    start(0, cb0, ab0, sem0)

    def pair(p, _):
        k0, k1 = 2 * p, 2 * p + 1
        start(k1, cb1, ab1, sem1)
        drain(k0, cb0, ab0, sem0)

        @pl.when(p < P2_BLOCKS // 2 - 1)
        def _():
            start(k0 + 2, cb0, ab0, sem0)

        drain(k1, cb1, ab1, sem1)
        return 0

    lax.fori_loop(0, P2_BLOCKS // 2, pair, 0)


def _rows_body(v_ref, rid_ref, rs_ref, rc_ref):
    i = pl.program_id(0)

    @pl.when(i == 0)
    def _():
        rs_ref[...] = jnp.zeros((CAMS_PAD, D), jnp.float32)
        rc_ref[...] = jnp.zeros((CAMS_PAD, D), jnp.float32)

    valid = (lax.broadcasted_iota(jnp.int32, (512, 1), 0) + i * 512) < NNZ
    v = jnp.where(valid, v_ref[...], 0.0)
    rid = rid_ref[...][0, 0]
    oh = (rid[:, None]
          == lax.broadcasted_iota(jnp.int32, (512, CAMS_PAD), 1))
    oh = jnp.where(valid, oh.astype(jnp.float32), 0.0)
    dn = (((0,), (0,)), ((), ()))
    rs_ref[...] += lax.dot_general(oh, v, dn)
    rc_ref[...] += lax.dot_general(oh, jnp.ones((512, D), jnp.float32), dn)


def _tables_body(cm_ref, rs_ref, rc_ref, wn_ref, wm_ref, wb_ref,
                 bsum_ref, bm_ref, a_ref, b_ref):
    rmask = lax.broadcasted_iota(jnp.int32, (CAMS_PAD, 1), 0) < N_CAMS
    rs = jnp.where(rmask, rs_ref[...], 0.0)
    g = jnp.sum(rs, axis=0, keepdims=True)
    go = jnp.dot(g, wb_ref[...], precision=_HI)
    const = 0.25 * (bsum_ref[...] + go * (1.0 / NNZ))
    rmean = rs / jnp.maximum(rc_ref[...], 1.0)
    bmat = 0.25 * (jnp.dot(rmean, wm_ref[...], precision=_HI) + bm_ref[...])
    b_ref[...] = jnp.where(rmask, bmat, 0.0)
    a_ref[...] = (0.25 * jnp.dot(cm_ref[...], wn_ref[...], precision=_HI)
                  + const)


def _final_body(v_ref, g_ref, rid_ref, wa_ref, b_ref, out_ref):
    rid = rid_ref[...][0, 0]
    oh = (rid[:, None]
          == lax.broadcasted_iota(jnp.int32, (512, CAMS_PAD), 1))
    oh = oh.astype(jnp.float32)
    out_ref[...] = (0.25 * jnp.dot(v_ref[...], wa_ref[...], precision=_HI)
                    + g_ref[...]
                    + jnp.dot(oh, b_ref[...]))


def kernel(values, row_idx, col_idx, W_all, b_all, W_n, b_n, W_m, b_m,
           W_both, b_both):
    f32 = jnp.float32
    row_idx = row_idx.astype(jnp.int32)
    col_idx = col_idx.astype(jnp.int32)

    rid3 = jnp.concatenate(
        [row_idx, jnp.full((NNZ_RPAD - NNZ,), ROW_DUMP, jnp.int32)]
    ).reshape(NNZ_BLOCKS, 1, 512)
    cidx2 = jnp.concatenate(
        [col_idx, jnp.zeros((NNZ_PAD - NNZ,), jnp.int32)]
    ).reshape(IDX_ROWS, 128)

    ccnt = jax.ops.segment_sum(jnp.ones((NNZ,), f32), col_idx,
                               num_segments=PTS_PAD)
    csum = jax.ops.segment_sum(values, col_idx, num_segments=PTS_PAD)
    cmean = csum / jnp.maximum(ccnt, 1.0)[:, None]

    row_sums, row_cntb = pl.pallas_call(
        _rows_body,
        grid=(NNZ_BLOCKS,),
        in_specs=[
            pl.BlockSpec((512, D), lambda i: (i, 0)),
            pl.BlockSpec((1, 1, 512), lambda i: (i, 0, 0)),
        ],
        out_specs=[
            pl.BlockSpec((CAMS_PAD, D), lambda i: (0, 0)),
            pl.BlockSpec((CAMS_PAD, D), lambda i: (0, 0)),
        ],
        out_shape=[
            jax.ShapeDtypeStruct((CAMS_PAD, D), f32),
            jax.ShapeDtypeStruct((CAMS_PAD, D), f32),
        ],
    )(values, rid3)

    bsum = (b_all + b_n + b_both).reshape(1, D)
    bm = b_m.reshape(1, D)

    a_tbl, b_tbl = pl.pallas_call(
        _tables_body,
        grid=(98,),
        in_specs=[
            pl.BlockSpec((512, D), lambda i: (i, 0)),
            pl.BlockSpec((CAMS_PAD, D), lambda i: (0, 0)),
            pl.BlockSpec((CAMS_PAD, D), lambda i: (0, 0)),
            pl.BlockSpec((D, D), lambda i: (0, 0)),
            pl.BlockSpec((D, D), lambda i: (0, 0)),
            pl.BlockSpec((D, D), lambda i: (0, 0)),
            pl.BlockSpec((1, D), lambda i: (0, 0)),
            pl.BlockSpec((1, D), lambda i: (0, 0)),
        ],
        out_specs=[
            pl.BlockSpec((512, D), lambda i: (i, 0)),
            pl.BlockSpec((CAMS_PAD, D), lambda i: (0, 0)),
        ],
        out_shape=[
            jax.ShapeDtypeStruct((PTS_PAD, D), f32),
            jax.ShapeDtypeStruct((CAMS_PAD, D), f32),
        ],
    )(cmean, row_sums, row_cntb, W_n, W_m, W_both, bsum, bm)

    gath = _gather_a(a_tbl, cidx2)

    out = pl.pallas_call(
        _final_body,
        grid=(NNZ_BLOCKS,),
        in_specs=[
            pl.BlockSpec((512, D), lambda i: (i, 0)),
            pl.BlockSpec((512, D), lambda i: (i, 0)),
            pl.BlockSpec((1, 1, 512), lambda i: (i, 0, 0)),
            pl.BlockSpec((D, D), lambda i: (0, 0)),
            pl.BlockSpec((CAMS_PAD, D), lambda i: (0, 0)),
        ],
        out_specs=pl.BlockSpec((512, D), lambda i: (i, 0)),
        out_shape=jax.ShapeDtypeStruct((NNZ, D), f32),
    )(values, gath, rid3, W_all, b_tbl)

    return out


# speedup vs baseline: 1.4160x; 1.0241x over previous
import functools

import jax
import jax.numpy as jnp
from jax import lax
from jax.experimental import pallas as pl
from jax.experimental.pallas import tpu as pltpu
from jax.experimental.pallas import tpu_sc as plsc

N_CAMS = 500
N_PTS = 50000
NNZ = 500000
D = 128

PTS_PAD = 50048
CAMS_PAD = 512
NNZ_BLOCKS = 977
NNZ_RPAD = NNZ_BLOCKS * 512
IDX_ROWS = 3968
NNZ_PAD = IDX_ROWS * 128
ROW_DUMP = 504

P2_ROWS_PER_TILE = IDX_ROWS // 32
P2_BLOCKS = 62

_MESH = plsc.VectorSubcoreMesh(core_axis_name="c", subcore_axis_name="s")
_HI = lax.Precision.HIGHEST


@functools.partial(
    pl.kernel,
    out_type=jax.ShapeDtypeStruct((NNZ_PAD, D), jnp.float32),
    mesh=_MESH,
    scratch_types=dict(
        cb0=pltpu.VMEM((2, 128), jnp.int32),
        cb1=pltpu.VMEM((2, 128), jnp.int32),
        ab0=pltpu.VMEM((256, D), jnp.float32),
        ab1=pltpu.VMEM((256, D), jnp.float32),
        sem0=pltpu.SemaphoreType.DMA,
        sem1=pltpu.SemaphoreType.DMA,
    ),
)
def _gather_a(a_tbl, cidx2, gath, *, cb0, cb1, ab0, ab1, sem0, sem1):
    c = lax.axis_index("c")
    s = lax.axis_index("s")
    w = s * 2 + c
    tile_row = w * P2_ROWS_PER_TILE

    def start(k, cb, ab, sem):
        pltpu.sync_copy(cidx2.at[pl.ds(tile_row + k * 2, 2)], cb)
        for j in range(2):
            pltpu.async_copy(a_tbl.at[cb.at[j]],
                             ab.at[pl.ds(j * 128, 128)], sem)

    def drain(k, cb, ab, sem):
        for j in range(2):
            pltpu.make_async_copy(a_tbl.at[cb.at[j]],
                                  ab.at[pl.ds(j * 128, 128)], sem).wait()
        pltpu.sync_copy(ab, gath.at[pl.ds((tile_row + k * 2) * 128, 256)])

    start(0, cb0, ab0, sem0)

    def pair(p, _):
        k0, k1 = 2 * p, 2 * p + 1
        start(k1, cb1, ab1, sem1)
        drain(k0, cb0, ab0, sem0)

        @pl.when(p < P2_BLOCKS // 2 - 1)
        def _():
            start(k0 + 2, cb0, ab0, sem0)

        drain(k1, cb1, ab1, sem1)
        return 0

    lax.fori_loop(0, P2_BLOCKS // 2, pair, 0)


def _rows_body(v_ref, rid_ref, rs_ref, rc_ref):
    i = pl.program_id(0)

    @pl.when(i == 0)
    def _():
        rs_ref[...] = jnp.zeros((CAMS_PAD, D), jnp.float32)
        rc_ref[...] = jnp.zeros((CAMS_PAD, D), jnp.float32)

    valid = (lax.broadcasted_iota(jnp.int32, (512, 1), 0) + i * 512) < NNZ
    v = jnp.where(valid, v_ref[...], 0.0)
    rid = rid_ref[...][0, 0]
    oh = (rid[:, None]
          == lax.broadcasted_iota(jnp.int32, (512, CAMS_PAD), 1))
    oh = jnp.where(valid, oh.astype(jnp.float32), 0.0)
    dn = (((0,), (0,)), ((), ()))
    rs_ref[...] += lax.dot_general(oh, v, dn)
    rc_ref[...] += lax.dot_general(oh, jnp.ones((512, D), jnp.float32), dn)


def _tables_body(cm_ref, rs_ref, rc_ref, wn_ref, wm_ref, wb_ref,
                 bsum_ref, bm_ref, a_ref, b_ref):
    rmask = lax.broadcasted_iota(jnp.int32, (CAMS_PAD, 1), 0) < N_CAMS
    rs = jnp.where(rmask, rs_ref[...], 0.0)
    g = jnp.sum(rs, axis=0, keepdims=True)
    go = jnp.dot(g, wb_ref[...], precision=_HI)
    const = 0.25 * (bsum_ref[...] + go * (1.0 / NNZ))
    rmean = rs / jnp.maximum(rc_ref[...], 1.0)
    bmat = 0.25 * (jnp.dot(rmean, wm_ref[...], precision=_HI) + bm_ref[...])
    b_ref[...] = jnp.where(rmask, bmat, 0.0)
    a_ref[...] = (0.25 * jnp.dot(cm_ref[...], wn_ref[...], precision=_HI)
                  + const)


def _final_body(v_ref, g_ref, rid_ref, wa_ref, b_ref, out_ref):
    rid = rid_ref[...][0, 0]
    oh = (rid[:, None]
          == lax.broadcasted_iota(jnp.int32, (512, CAMS_PAD), 1))
    oh = oh.astype(jnp.float32)
    out_ref[...] = (0.25 * jnp.dot(v_ref[...], wa_ref[...])
                    + g_ref[...]
                    + jnp.dot(oh, b_ref[...]))


def kernel(values, row_idx, col_idx, W_all, b_all, W_n, b_n, W_m, b_m,
           W_both, b_both):
    f32 = jnp.float32
    row_idx = row_idx.astype(jnp.int32)
    col_idx = col_idx.astype(jnp.int32)

    rid3 = jnp.concatenate(
        [row_idx, jnp.full((NNZ_RPAD - NNZ,), ROW_DUMP, jnp.int32)]
    ).reshape(NNZ_BLOCKS, 1, 512)
    cidx2 = jnp.concatenate(
        [col_idx, jnp.zeros((NNZ_PAD - NNZ,), jnp.int32)]
    ).reshape(IDX_ROWS, 128)

    ccnt = jax.ops.segment_sum(jnp.ones((NNZ,), f32), col_idx,
                               num_segments=PTS_PAD)
    csum = jax.ops.segment_sum(values, col_idx, num_segments=PTS_PAD)
    cmean = csum / jnp.maximum(ccnt, 1.0)[:, None]

    row_sums, row_cntb = pl.pallas_call(
        _rows_body,
        grid=(NNZ_BLOCKS,),
        in_specs=[
            pl.BlockSpec((512, D), lambda i: (i, 0)),
            pl.BlockSpec((1, 1, 512), lambda i: (i, 0, 0)),
        ],
        out_specs=[
            pl.BlockSpec((CAMS_PAD, D), lambda i: (0, 0)),
            pl.BlockSpec((CAMS_PAD, D), lambda i: (0, 0)),
        ],
        out_shape=[
            jax.ShapeDtypeStruct((CAMS_PAD, D), f32),
            jax.ShapeDtypeStruct((CAMS_PAD, D), f32),
        ],
    )(values, rid3)

    bsum = (b_all + b_n + b_both).reshape(1, D)
    bm = b_m.reshape(1, D)

    a_tbl, b_tbl = pl.pallas_call(
        _tables_body,
        grid=(98,),
        in_specs=[
            pl.BlockSpec((512, D), lambda i: (i, 0)),
            pl.BlockSpec((CAMS_PAD, D), lambda i: (0, 0)),
            pl.BlockSpec((CAMS_PAD, D), lambda i: (0, 0)),
            pl.BlockSpec((D, D), lambda i: (0, 0)),
            pl.BlockSpec((D, D), lambda i: (0, 0)),
            pl.BlockSpec((D, D), lambda i: (0, 0)),
            pl.BlockSpec((1, D), lambda i: (0, 0)),
            pl.BlockSpec((1, D), lambda i: (0, 0)),
        ],
        out_specs=[
            pl.BlockSpec((512, D), lambda i: (i, 0)),
            pl.BlockSpec((CAMS_PAD, D), lambda i: (0, 0)),
        ],
        out_shape=[
            jax.ShapeDtypeStruct((PTS_PAD, D), f32),
            jax.ShapeDtypeStruct((CAMS_PAD, D), f32),
        ],
    )(cmean, row_sums, row_cntb, W_n, W_m, W_both, bsum, bm)

    gath = _gather_a(a_tbl, cidx2)

    out = pl.pallas_call(
        _final_body,
        grid=(NNZ_BLOCKS,),
        in_specs=[
            pl.BlockSpec((512, D), lambda i: (i, 0)),
            pl.BlockSpec((512, D), lambda i: (i, 0)),
            pl.BlockSpec((1, 1, 512), lambda i: (i, 0, 0)),
            pl.BlockSpec((D, D), lambda i: (0, 0)),
            pl.BlockSpec((CAMS_PAD, D), lambda i: (0, 0)),
        ],
        out_specs=pl.BlockSpec((512, D), lambda i: (i, 0)),
        out_shape=jax.ShapeDtypeStruct((NNZ, D), f32),
    )(values, gath, rid3, W_all, b_tbl)

    return out
